# baseline (speedup 1.0000x reference)
import functools

import jax
import jax.numpy as jnp
from jax import lax
from jax.experimental import pallas as pl
from jax.experimental.pallas import tpu as pltpu
from jax.experimental.pallas import tpu_sc as plsc

_EPS = 1e-5

_NC = 2
_NS = 16
_NW = _NC * _NS
_RB_SC = 80
_N_SC = 20000


def _seg_kernel(batch_ref, x_ref, sums_ref, counts_ref):
    i = pl.program_id(0)

    @pl.when(i == 0)
    def _init():
        sums_ref[...] = jnp.zeros_like(sums_ref)
        counts_ref[...] = jnp.zeros_like(counts_ref)

    b = batch_ref[0]
    B = sums_ref.shape[0]
    RB = b.shape[-1]
    oh = (lax.broadcasted_iota(jnp.int32, (B, RB), 0)
          == jnp.broadcast_to(b, (B, RB))).astype(jnp.float32)
    sums_ref[...] += lax.dot(oh, x_ref[...], preferred_element_type=jnp.float32)
    counts_ref[...] += jnp.sum(oh, axis=1)[None, :]


def _mlp_kernel(n_rows, sums_ref, counts_ref, vn_ref,
                Wn_ref, bn_ref, gn_ref, ben_ref,
                Wv_ref, bv_ref, gv_ref, bev_ref,
                z_ref, vn_out_ref):
    counts = counts_ref[0, :]
    cnt = counts[:, None]
    ntv = sums_ref[...] / jnp.where(cnt > 0, cnt, 1.0)
    h = lax.dot_general(ntv, Wn_ref[...], (((1,), (1,)), ((), ())),
                        preferred_element_type=jnp.float32) + bn_ref[...]
    mu = jnp.mean(h, axis=0)
    var = jnp.mean((h - mu[None, :]) ** 2, axis=0)
    h = gn_ref[...] * (h - mu[None, :]) * lax.rsqrt(var[None, :] + _EPS) + ben_ref[...]
    h = jnp.maximum(h, 0.0)
    vn_out = vn_ref[...] + h
    vn_out_ref[...] = vn_out
    y = lax.dot_general(vn_out, Wv_ref[...], (((1,), (1,)), ((), ())),
                        preferred_element_type=jnp.float32) + bv_ref[...]
    w = (counts / jnp.float32(n_rows))[:, None]
    mu2 = jnp.sum(w * y, axis=0)
    var2 = jnp.sum(w * (y - mu2[None, :]) ** 2, axis=0)
    z = gv_ref[...] * (y - mu2[None, :]) * lax.rsqrt(var2[None, :] + _EPS) + bev_ref[...]
    z_ref[...] = jnp.maximum(z, 0.0)


def _bcast_kernel(batch_ref, x_ref, z_ref, out_ref):
    b = batch_ref[0]
    B = z_ref.shape[0]
    RC = b.shape[-1]
    oh = (lax.broadcasted_iota(jnp.int32, (B, RC), 0)
          == jnp.broadcast_to(b, (B, RC))).astype(jnp.float32)
    gathered = lax.dot_general(oh, z_ref[...], (((0,), (0,)), ((), ())),
                               preferred_element_type=jnp.float32)
    out_ref[...] = x_ref[...] + gathered


def _sc_bcast_kernel(n_rows, d, x_hbm, batch_hbm, z_hbm, out_hbm,
                     xblk, zblk, idx_v, sem):
    c = lax.axis_index("c")
    s = lax.axis_index("s")
    w = s * _NC + c
    nblk = n_rows // _RB_SC
    nblk_w = (nblk + _NW - 1) // _NW

    def body(j, carry):
        blk = w + j * _NW

        @pl.when(blk < nblk)
        def _do():
            row = blk * _RB_SC
            pltpu.sync_copy(batch_hbm.at[pl.ds(row, _RB_SC)], idx_v)
            pltpu.async_copy(z_hbm.at[idx_v], zblk, sem).wait()
            pltpu.sync_copy(x_hbm.at[pl.ds(row, _RB_SC)], xblk)
            for r in range(_RB_SC):
                for k in range(d // 16):
                    sl = pl.ds(k * 16, 16)
                    xblk[r, sl] = xblk[r, sl] + zblk[r, sl]
            pltpu.sync_copy(xblk, out_hbm.at[pl.ds(row, _RB_SC)])

        return carry

    lax.fori_loop(0, nblk_w, body, 0)


def _sc_bcast(x_head, batch_head, z):
    n, d = x_head.shape
    mesh = plsc.VectorSubcoreMesh(core_axis_name="c", subcore_axis_name="s",
                                  num_cores=_NC, num_subcores=_NS)
    f = pl.kernel(
        functools.partial(_sc_bcast_kernel, n, d),
        out_type=jax.ShapeDtypeStruct((n, d), jnp.float32),
        mesh=mesh,
        scratch_types=[
            pltpu.VMEM((_RB_SC, d), jnp.float32),
            pltpu.VMEM((_RB_SC, d), jnp.float32),
            pltpu.VMEM((_RB_SC,), jnp.int32),
            pltpu.SemaphoreType.DMA,
        ],
    )
    return f(x_head, batch_head, z)


def kernel(x, vn_embedding, batch, W_vn2node, b_vn2node, g_vn2node, be_vn2node,
           W_node2vn, b_node2vn, g_node2vn, be_node2vn):
    N, D = x.shape
    B = vn_embedding.shape[0]
    batch = batch.astype(jnp.int32)

    RB = 2000
    nblk = N // RB
    batch3 = batch.reshape(nblk, 1, RB)

    sums, counts = pl.pallas_call(
        _seg_kernel,
        grid=(nblk,),
        in_specs=[
            pl.BlockSpec((1, 1, RB), lambda i: (i, 0, 0)),
            pl.BlockSpec((RB, D), lambda i: (i, 0)),
        ],
        out_specs=[
            pl.BlockSpec((B, D), lambda i: (0, 0)),
            pl.BlockSpec((1, B), lambda i: (0, 0)),
        ],
        out_shape=[
            jax.ShapeDtypeStruct((B, D), jnp.float32),
            jax.ShapeDtypeStruct((1, B), jnp.float32),
        ],
    )(batch3, x)

    row = lambda v: v.reshape(1, D)
    z, vn_out = pl.pallas_call(
        functools.partial(_mlp_kernel, N),
        out_shape=[
            jax.ShapeDtypeStruct((B, D), jnp.float32),
            jax.ShapeDtypeStruct((B, D), jnp.float32),
        ],
    )(sums, counts, vn_embedding,
      W_node2vn, row(b_node2vn), row(g_node2vn), row(be_node2vn),
      W_vn2node, row(b_vn2node), row(g_vn2node), row(be_vn2node))

    n_sc = _N_SC
    out_head = _sc_bcast(x[:n_sc], batch[:n_sc], z)

    n_tc = N - n_sc
    RC = 2000
    nblk_c = n_tc // RC
    assert nblk_c * RC == n_tc
    batch3c = batch[n_sc:].reshape(nblk_c, 1, RC)
    out_tail = pl.pallas_call(
        _bcast_kernel,
        grid=(nblk_c,),
        in_specs=[
            pl.BlockSpec((1, 1, RC), lambda i: (i, 0, 0)),
            pl.BlockSpec((RC, D), lambda i: (i, 0)),
            pl.BlockSpec((B, D), lambda i: (0, 0)),
        ],
        out_specs=pl.BlockSpec((RC, D), lambda i: (i, 0)),
        out_shape=jax.ShapeDtypeStruct((n_tc, D), jnp.float32),
    )(batch3c, x[n_sc:], z)

    x_out = jnp.concatenate([out_head, out_tail], axis=0)
    return (x_out, vn_out)


# speedup vs baseline: 1.9102x; 1.9102x over previous
import functools

import jax
import jax.numpy as jnp
from jax import lax
from jax.experimental import pallas as pl
from jax.experimental.pallas import tpu as pltpu
from jax.experimental.pallas import tpu_sc as plsc

_EPS = 1e-5

_NC = 2
_NS = 16
_NW = _NC * _NS
_RB_SC = 80
_N_SC = 20000


def _sc_seg_kernel(n_rows, d, x_hbm, batch_hbm, zs_hbm, zc_hbm,
                   sums_hbm, cnts_hbm,
                   xblk, idx_v, acc, cnt):
    c = lax.axis_index("c")
    s = lax.axis_index("s")
    w = s * _NC + c
    nblk = n_rows // _RB_SC
    nblk_w = (nblk + _NW - 1) // _NW

    pltpu.sync_copy(zs_hbm, acc)
    pltpu.sync_copy(zc_hbm, cnt)
    ones16 = jnp.ones((16,), jnp.float32)

    def grp_body(g, carry):
        v = idx_v[pl.ds(g * 16, 16)]
        for l in range(16):
            b = jnp.squeeze(lax.slice(v, (l,), (l + 1,)))
            r = g * 16 + l
            cnt[b, :] = cnt[b, :] + ones16
            for k in range(d // 16):
                sl = pl.ds(k * 16, 16)
                acc[b, sl] = acc[b, sl] + xblk[r, sl]
        return carry

    def body(j, carry):
        blk = w + j * _NW

        @pl.when(blk < nblk)
        def _do():
            row = blk * _RB_SC
            pltpu.sync_copy(x_hbm.at[pl.ds(row, _RB_SC)], xblk)
            pltpu.sync_copy(batch_hbm.at[pl.ds(row, _RB_SC)], idx_v)
            lax.fori_loop(0, _RB_SC // 16, grp_body, 0)

        return carry

    lax.fori_loop(0, nblk_w, body, 0)
    pltpu.sync_copy(acc, sums_hbm.at[w])
    pltpu.sync_copy(cnt, cnts_hbm.at[w])


def _sc_segment_sums(x_head, batch_head):
    n, d = x_head.shape
    B = 64
    mesh = plsc.VectorSubcoreMesh(core_axis_name="c", subcore_axis_name="s",
                                  num_cores=_NC, num_subcores=_NS)
    zs = jnp.zeros((B, d), jnp.float32)
    zc = jnp.zeros((B, 16), jnp.float32)
    f = pl.kernel(
        functools.partial(_sc_seg_kernel, n, d),
        out_type=(
            jax.ShapeDtypeStruct((_NW, B, d), jnp.float32),
            jax.ShapeDtypeStruct((_NW, B, 16), jnp.float32),
        ),
        mesh=mesh,
        scratch_types=[
            pltpu.VMEM((_RB_SC, d), jnp.float32),
            pltpu.VMEM((_RB_SC,), jnp.int32),
            pltpu.VMEM((B, d), jnp.float32),
            pltpu.VMEM((B, 16), jnp.float32),
        ],
    )
    return f(x_head, batch_head, zs, zc)


def _seg_kernel(batch_ref, x_ref, sums_ref, counts_ref):
    i = pl.program_id(0)

    @pl.when(i == 0)
    def _init():
        sums_ref[...] = jnp.zeros_like(sums_ref)
        counts_ref[...] = jnp.zeros_like(counts_ref)

    b = batch_ref[0]
    B = sums_ref.shape[0]
    RB = b.shape[-1]
    oh = (lax.broadcasted_iota(jnp.int32, (B, RB), 0)
          == jnp.broadcast_to(b, (B, RB))).astype(jnp.float32)
    sums_ref[...] += lax.dot(oh, x_ref[...], preferred_element_type=jnp.float32)
    counts_ref[...] += jnp.sum(oh, axis=1)[None, :]


def _mlp_kernel(n_rows, sums_ref, counts_ref, sc_sums_ref, sc_cnts_ref, vn_ref,
                Wn_ref, bn_ref, gn_ref, ben_ref,
                Wv_ref, bv_ref, gv_ref, bev_ref,
                z_ref, vn_out_ref):
    counts = counts_ref[0, :] + jnp.sum(sc_cnts_ref[...], axis=0)
    cnt = counts[:, None]
    sums = sums_ref[...] + jnp.sum(sc_sums_ref[...], axis=0)
    ntv = sums / jnp.where(cnt > 0, cnt, 1.0)
    h = lax.dot_general(ntv, Wn_ref[...], (((1,), (1,)), ((), ())),
                        preferred_element_type=jnp.float32) + bn_ref[...]
    mu = jnp.mean(h, axis=0)
    var = jnp.mean((h - mu[None, :]) ** 2, axis=0)
    h = gn_ref[...] * (h - mu[None, :]) * lax.rsqrt(var[None, :] + _EPS) + ben_ref[...]
    h = jnp.maximum(h, 0.0)
    vn_out = vn_ref[...] + h
    vn_out_ref[...] = vn_out
    y = lax.dot_general(vn_out, Wv_ref[...], (((1,), (1,)), ((), ())),
                        preferred_element_type=jnp.float32) + bv_ref[...]
    w = (counts / jnp.float32(n_rows))[:, None]
    mu2 = jnp.sum(w * y, axis=0)
    var2 = jnp.sum(w * (y - mu2[None, :]) ** 2, axis=0)
    z = gv_ref[...] * (y - mu2[None, :]) * lax.rsqrt(var2[None, :] + _EPS) + bev_ref[...]
    z_ref[...] = jnp.maximum(z, 0.0)


def _bcast_kernel(batch_ref, x_ref, z_ref, out_ref):
    b = batch_ref[0]
    B = z_ref.shape[0]
    RC = b.shape[-1]
    oh = (lax.broadcasted_iota(jnp.int32, (B, RC), 0)
          == jnp.broadcast_to(b, (B, RC))).astype(jnp.float32)
    gathered = lax.dot_general(oh, z_ref[...], (((0,), (0,)), ((), ())),
                               preferred_element_type=jnp.float32)
    out_ref[...] = x_ref[...] + gathered


def kernel(x, vn_embedding, batch, W_vn2node, b_vn2node, g_vn2node, be_vn2node,
           W_node2vn, b_node2vn, g_node2vn, be_node2vn):
    N, D = x.shape
    B = vn_embedding.shape[0]
    batch = batch.astype(jnp.int32)

    n_sc = _N_SC
    sc_sums, sc_cnts = _sc_segment_sums(x[:n_sc], batch[:n_sc])

    RB = 2000
    nblk = N // RB
    blk0 = n_sc // RB
    batch3 = batch.reshape(nblk, 1, RB)
    sums, counts = pl.pallas_call(
        _seg_kernel,
        grid=(nblk - blk0,),
        in_specs=[
            pl.BlockSpec((1, 1, RB), lambda i: (i + blk0, 0, 0)),
            pl.BlockSpec((RB, D), lambda i: (i + blk0, 0)),
        ],
        out_specs=[
            pl.BlockSpec((B, D), lambda i: (0, 0)),
            pl.BlockSpec((1, B), lambda i: (0, 0)),
        ],
        out_shape=[
            jax.ShapeDtypeStruct((B, D), jnp.float32),
            jax.ShapeDtypeStruct((1, B), jnp.float32),
        ],
    )(batch3, x)

    row = lambda v: v.reshape(1, D)
    z, vn_out = pl.pallas_call(
        functools.partial(_mlp_kernel, N),
        out_shape=[
            jax.ShapeDtypeStruct((B, D), jnp.float32),
            jax.ShapeDtypeStruct((B, D), jnp.float32),
        ],
    )(sums, counts, sc_sums, sc_cnts[:, :, 0], vn_embedding,
      W_node2vn, row(b_node2vn), row(g_node2vn), row(be_node2vn),
      W_vn2node, row(b_vn2node), row(g_vn2node), row(be_vn2node))

    x_out = pl.pallas_call(
        _bcast_kernel,
        grid=(nblk,),
        in_specs=[
            pl.BlockSpec((1, 1, RB), lambda i: (i, 0, 0)),
            pl.BlockSpec((RB, D), lambda i: (i, 0)),
            pl.BlockSpec((B, D), lambda i: (0, 0)),
        ],
        out_specs=pl.BlockSpec((RB, D), lambda i: (i, 0)),
        out_shape=jax.ShapeDtypeStruct((N, D), jnp.float32),
    )(batch3, x, z)

    return (x_out, vn_out)


# speedup vs baseline: 3.5103x; 1.8377x over previous
import functools

import jax
import jax.numpy as jnp
from jax import lax
from jax.experimental import pallas as pl
from jax.experimental.pallas import tpu as pltpu
from jax.experimental.pallas import tpu_sc as plsc

_EPS = 1e-5

_NC = 2
_NS = 16
_NW = _NC * _NS
_RB_SC = 80
_BLKS_W = 8
_N_SC = _NW * _BLKS_W * _RB_SC
_RB_TC = 1280


def _sc_seg_kernel(d, x_hbm, batch_hbm, zs_hbm, zc_hbm,
                   sums_hbm, cnts_hbm,
                   xblk, idx_v, acc, cnt):
    c = lax.axis_index("c")
    s = lax.axis_index("s")
    w = s * _NC + c
    base = w * (_BLKS_W * _RB_SC)
    nk = d // 16

    pltpu.sync_copy(zs_hbm, acc)
    pltpu.sync_copy(zc_hbm, cnt)
    zero16 = jnp.zeros((16,), jnp.float32)

    def grp_body(g, carry):
        b_prev, c_run, regs = carry
        v = idx_v[pl.ds(g * 16, 16)]
        for l in range(16):
            b = jnp.squeeze(lax.slice(v, (l,), (l + 1,)))
            r = g * 16 + l
            same = b == b_prev

            @pl.when(jnp.logical_not(same))
            def _flush(b_prev=b_prev, c_run=c_run, regs=regs):
                cnt[b_prev, :] = cnt[b_prev, :] + (zero16 + c_run)
                for k in range(nk):
                    sl = pl.ds(k * 16, 16)
                    acc[b_prev, sl] = acc[b_prev, sl] + regs[k]

            regs = tuple(
                jnp.where(same, regs[k], 0.0) + xblk[r, pl.ds(k * 16, 16)]
                for k in range(nk))
            c_run = jnp.where(same, c_run + 1.0, 1.0)
            b_prev = b
        return b_prev, c_run, regs

    def blk_body(j, carry):
        row = base + j * _RB_SC
        pltpu.sync_copy(x_hbm.at[pl.ds(row, _RB_SC)], xblk)
        pltpu.sync_copy(batch_hbm.at[pl.ds(row, _RB_SC)], idx_v)
        return lax.fori_loop(0, _RB_SC // 16, grp_body, carry)

    init = (jnp.int32(0), jnp.float32(0.0),
            tuple(zero16 for _ in range(nk)))
    b_prev, c_run, regs = lax.fori_loop(0, _BLKS_W, blk_body, init)
    cnt[b_prev, :] = cnt[b_prev, :] + (zero16 + c_run)
    for k in range(nk):
        sl = pl.ds(k * 16, 16)
        acc[b_prev, sl] = acc[b_prev, sl] + regs[k]

    pltpu.sync_copy(acc, sums_hbm.at[w])
    pltpu.sync_copy(cnt, cnts_hbm.at[w])


def _sc_segment_sums(x, batch_i32):
    d = x.shape[1]
    B = 64
    mesh = plsc.VectorSubcoreMesh(core_axis_name="c", subcore_axis_name="s",
                                  num_cores=_NC, num_subcores=_NS)
    zs = jnp.zeros((B, d), jnp.float32)
    zc = jnp.zeros((B, 16), jnp.float32)
    f = pl.kernel(
        functools.partial(_sc_seg_kernel, d),
        out_type=(
            jax.ShapeDtypeStruct((_NW, B, d), jnp.float32),
            jax.ShapeDtypeStruct((_NW, B, 16), jnp.float32),
        ),
        mesh=mesh,
        scratch_types=[
            pltpu.VMEM((_RB_SC, d), jnp.float32),
            pltpu.VMEM((_RB_SC,), jnp.int32),
            pltpu.VMEM((B, d), jnp.float32),
            pltpu.VMEM((B, 16), jnp.float32),
        ],
    )
    return f(x, batch_i32, zs, zc)


def _seg_kernel(batch_ref, x_ref, sums_ref, counts_ref):
    i = pl.program_id(0)

    @pl.when(i == 0)
    def _init():
        sums_ref[...] = jnp.zeros_like(sums_ref)
        counts_ref[...] = jnp.zeros_like(counts_ref)

    b = batch_ref[0]
    B = sums_ref.shape[0]
    RB = b.shape[-1]
    oh = (lax.broadcasted_iota(jnp.int32, (B, RB), 0)
          == jnp.broadcast_to(b, (B, RB))).astype(jnp.float32)
    sums_ref[...] += lax.dot(oh, x_ref[...], preferred_element_type=jnp.float32)
    counts_ref[...] += jnp.sum(oh, axis=1)[None, :]


def _mlp_kernel(n_rows, sums_ref, counts_ref, sc_sums_ref, sc_cnts_ref, vn_ref,
                Wn_ref, bn_ref, gn_ref, ben_ref,
                Wv_ref, bv_ref, gv_ref, bev_ref,
                z_ref, vn_out_ref):
    counts = counts_ref[0, :] + jnp.sum(sc_cnts_ref[...], axis=0)
    cnt = counts[:, None]
    sums = sums_ref[...] + jnp.sum(sc_sums_ref[...], axis=0)
    ntv = sums / jnp.where(cnt > 0, cnt, 1.0)
    h = lax.dot_general(ntv, Wn_ref[...], (((1,), (1,)), ((), ())),
                        preferred_element_type=jnp.float32) + bn_ref[...]
    mu = jnp.mean(h, axis=0)
    var = jnp.mean((h - mu[None, :]) ** 2, axis=0)
    h = gn_ref[...] * (h - mu[None, :]) * lax.rsqrt(var[None, :] + _EPS) + ben_ref[...]
    h = jnp.maximum(h, 0.0)
    vn_out = vn_ref[...] + h
    vn_out_ref[...] = vn_out
    y = lax.dot_general(vn_out, Wv_ref[...], (((1,), (1,)), ((), ())),
                        preferred_element_type=jnp.float32) + bv_ref[...]
    w = (counts / jnp.float32(n_rows))[:, None]
    mu2 = jnp.sum(w * y, axis=0)
    var2 = jnp.sum(w * (y - mu2[None, :]) ** 2, axis=0)
    z = gv_ref[...] * (y - mu2[None, :]) * lax.rsqrt(var2[None, :] + _EPS) + bev_ref[...]
    z_ref[...] = jnp.maximum(z, 0.0)


def _bcast_kernel(batch_ref, x_ref, z_ref, out_ref):
    b = batch_ref[0]
    B = z_ref.shape[0]
    RC = b.shape[-1]
    oh = (lax.broadcasted_iota(jnp.int32, (B, RC), 0)
          == jnp.broadcast_to(b, (B, RC))).astype(jnp.float32)
    gathered = lax.dot_general(oh, z_ref[...], (((0,), (0,)), ((), ())),
                               preferred_element_type=jnp.float32)
    out_ref[...] = x_ref[...] + gathered


def kernel(x, vn_embedding, batch, W_vn2node, b_vn2node, g_vn2node, be_vn2node,
           W_node2vn, b_node2vn, g_node2vn, be_node2vn):
    N, D = x.shape
    B = vn_embedding.shape[0]
    batch = batch.astype(jnp.int32)

    sc_sums, sc_cnts = _sc_segment_sums(x, batch)

    RB = _RB_TC
    nblk = -(-N // RB)
    batch_pad = jnp.concatenate(
        [batch, jnp.full((nblk * RB - N,), B, jnp.int32)])
    batch3 = batch_pad.reshape(nblk, 1, RB)
    blk0 = _N_SC // RB

    sums, counts = pl.pallas_call(
        _seg_kernel,
        grid=(nblk - blk0,),
        in_specs=[
            pl.BlockSpec((1, 1, RB), lambda i: (i + blk0, 0, 0)),
            pl.BlockSpec((RB, D), lambda i: (i + blk0, 0)),
        ],
        out_specs=[
            pl.BlockSpec((B, D), lambda i: (0, 0)),
            pl.BlockSpec((1, B), lambda i: (0, 0)),
        ],
        out_shape=[
            jax.ShapeDtypeStruct((B, D), jnp.float32),
            jax.ShapeDtypeStruct((1, B), jnp.float32),
        ],
    )(batch3, x)

    row = lambda v: v.reshape(1, D)
    z, vn_out = pl.pallas_call(
        functools.partial(_mlp_kernel, N),
        out_shape=[
            jax.ShapeDtypeStruct((B, D), jnp.float32),
            jax.ShapeDtypeStruct((B, D), jnp.float32),
        ],
    )(sums, counts, sc_sums, sc_cnts[:, :, 0], vn_embedding,
      W_node2vn, row(b_node2vn), row(g_node2vn), row(be_node2vn),
      W_vn2node, row(b_vn2node), row(g_vn2node), row(be_vn2node))

    x_out = pl.pallas_call(
        _bcast_kernel,
        grid=(nblk,),
        in_specs=[
            pl.BlockSpec((1, 1, RB), lambda i: (i, 0, 0)),
            pl.BlockSpec((RB, D), lambda i: (i, 0)),
            pl.BlockSpec((B, D), lambda i: (0, 0)),
        ],
        out_specs=pl.BlockSpec((RB, D), lambda i: (i, 0)),
        out_shape=jax.ShapeDtypeStruct((N, D), jnp.float32),
    )(batch3, x, z)

    return (x_out, vn_out)
